# baseline (speedup 1.0000x reference)
import functools

import jax
import jax.numpy as jnp
from jax import lax
from jax.experimental import pallas as pl
from jax.experimental.pallas import tpu as pltpu
from jax.experimental.pallas import tpu_sc as plsc

_N = 10000
_E = 160000
_D = 256
_NN = 500
_K = 256
_BATCH = 256

_NC, _NS, _L = 2, 16, 16
_NP = 10240
_ECH = 1280
_EP = _ECH * 128
_PTC = _ECH // _NS
_RT = _NP // _NS
_BLK = 2000
_NB = _N // _BLK
_RTO = _N // _NS

@functools.cache
def _sc_mesh():
    return plsc.VectorSubcoreMesh(core_axis_name="c", subcore_axis_name="s",
                                  num_cores=_NC, num_subcores=_NS)


def _nr_rsqrt(x):
    i = plsc.bitcast(x, jnp.int32)
    i = jnp.int32(0x5F3759DF) - (i >> 1)
    y = plsc.bitcast(i, jnp.float32)
    for _ in range(3):
        y = y * (1.5 - 0.5 * x * y * y)
    return y


def _sc_prep_body(src2d, dst2d, ew2d, dinv_out, norm_out,
                  degt, srct, dstt, ewt, rbuf, dbuf, dinvt, nrmt,
                  slab, sdinv):
    c = lax.axis_index("c")
    s = lax.axis_index("s")
    pltpu.sync_copy(dst2d.at[pl.ds(s * _PTCH, _PTCH)], dstt)
    pltpu.sync_copy(ew2d.at[pl.ds(s * _PTCH, _PTCH)], ewt)
    pltpu.sync_copy(src2d.at[pl.ds(s * _PTCH, _PTCH)], srct)

    def zero(i, _):
        degt[pl.ds(i * 16, 16)] = jnp.zeros((16,), jnp.float32)
        return 0
    lax.fori_loop(0, _NP // 16, zero, 0)

    def scat(k, _):
        for g in range(_CW // 16):
            d16 = dstt[k, pl.ds(g * 16, 16)]
            w16 = ewt[k, pl.ds(g * 16, 16)]
            plsc.addupdate_scatter(degt, [d16], w16)
        return 0
    lax.fori_loop(0, _PTCH, scat, 0)

    pltpu.sync_copy(degt, slab.at[s])
    plsc.subcore_barrier()

    r0 = s * _RT
    for p in range(_NS):
        pltpu.sync_copy(slab.at[p, pl.ds(r0, _RT)], rbuf.at[p])

    def red(v, _):
        acc = rbuf[0, pl.ds(v * 16, 16)]
        for p in range(1, _NS):
            acc = acc + rbuf[p, pl.ds(v * 16, 16)]
        acc = acc + 1.0
        dbuf[pl.ds(v * 16, 16)] = _nr_rsqrt(acc)
        return 0
    lax.fori_loop(0, _RT // 16, red, 0)

    pltpu.sync_copy(dbuf, sdinv.at[pl.ds(r0, _RT)])

    @pl.when(c == 0)
    def _():
        pltpu.sync_copy(dbuf, dinv_out.at[pl.ds(r0, _RT)])

    plsc.subcore_barrier()
    pltpu.sync_copy(sdinv, dinvt)

    def nrm(k, _):
        for g in range(_CW // 16):
            s16 = srct[k, pl.ds(g * 16, 16)]
            d16 = dstt[k, pl.ds(g * 16, 16)]
            w16 = ewt[k, pl.ds(g * 16, 16)]
            a = plsc.load_gather(dinvt, [s16])
            b = plsc.load_gather(dinvt, [d16])
            nrmt[k, pl.ds(g * 16, 16)] = a * w16 * b
        return 0
    lax.fori_loop(0, _PTCH, nrm, 0)

    @pl.when(c == 0)
    def _():
        pltpu.sync_copy(nrmt, norm_out.at[pl.ds(s * _PTCH, _PTCH)])


@functools.cache
def _sc_prep():
    return functools.partial(
        pl.kernel,
        out_type=(jax.ShapeDtypeStruct((_NP,), jnp.float32),
                  jax.ShapeDtypeStruct((_NCH, _CW), jnp.float32)),
        mesh=_sc_mesh(),
        compiler_params=pltpu.CompilerParams(needs_layout_passes=False),
        scratch_types=[
            pltpu.VMEM((_NP,), jnp.float32),
            pltpu.VMEM((_PTCH, _CW), jnp.int32),
            pltpu.VMEM((_PTCH, _CW), jnp.int32),
            pltpu.VMEM((_PTCH, _CW), jnp.float32),
            pltpu.VMEM((_NS, _RT), jnp.float32),
            pltpu.VMEM((_RT,), jnp.float32),
            pltpu.VMEM((_NP,), jnp.float32),
            pltpu.VMEM((_PTCH, _CW), jnp.float32),
            pltpu.VMEM_SHARED((_NS, _NP), jnp.float32),
            pltpu.VMEM_SHARED((_NP,), jnp.float32),
        ],
    )(_sc_prep_body)


_CW = 80
_NCH = _EP // _CW
_PTCH = _NCH // _NS
_SGC = 16
_NST = _PTCH // _SGC


def _sc_pass_body(src2d, dst2d, nrm2d, xlh, acc_out,
                  srct, dstt, nrmt, gb0, gb1, gb2, gb3,
                  accS, sg0, sg1, sg2, sg3, ss0, ss1, ss2, ss3):
    c = lax.axis_index("c")
    s = lax.axis_index("s")
    gbs = (gb0, gb1, gb2, gb3)
    sgs = (sg0, sg1, sg2, sg3)
    sss = (ss0, ss1, ss2, ss3)

    def zero(i, _):
        for j in range(8):
            gb0[i, pl.ds(j * 16, 16)] = jnp.zeros((16,), jnp.float32)
        return 0
    lax.fori_loop(0, _CW, zero, 0)
    r0 = s * _RT
    for t in range(_RT // _CW):
        pltpu.sync_copy(gb0, accS.at[pl.ds(r0 + t * _CW, _CW)])
    plsc.subcore_barrier()

    xl = xlh.at[c]

    def scale(buf, k):
        def grp(g, _):
            n16 = nrmt[k, pl.ds(g * 16, 16)]
            for e in range(16):
                nb = n16.at[jnp.full((16,), e, jnp.int32)].get(
                    mode="promise_in_bounds")
                row = g * 16 + e
                for j in range(8):
                    buf[row, pl.ds(j * 16, 16)] = (
                        buf[row, pl.ds(j * 16, 16)] * nb)
            return 0
        lax.fori_loop(0, _CW // 16, grp, 0)

    def gath(k, b):
        pltpu.async_copy(xl.at[srct.at[k]], gbs[b], sgs[b])

    def wait_g(b):
        pltpu.make_async_copy(xl.at[srct.at[0]], gbs[b], sgs[b]).wait()

    def scat(k, b):
        pltpu.async_copy(gbs[b], accS.at[dstt.at[k]], sss[b], add=True)

    def wait_s(b):
        pltpu.make_async_copy(gbs[b], accS.at[dstt.at[0]], sss[b]).wait()

    def proc(k, b, p, first=False):
        if not first:
            wait_s(p)
        kn = k + 3 if isinstance(k, int) and k + 3 < _SGC else (
            jnp.minimum(k + 3, _SGC - 1))
        gath(kn, p)
        wait_g(b)
        scale(gbs[b], k)
        scat(k, b)

    def stage(st, _):
        g0 = s * _PTCH + st * _SGC
        pltpu.sync_copy(src2d.at[pl.ds(g0, _SGC)], srct)
        pltpu.sync_copy(dst2d.at[pl.ds(g0, _SGC)], dstt)
        pltpu.sync_copy(nrm2d.at[pl.ds(g0, _SGC)], nrmt)
        gath(0, 0)
        gath(1, 1)
        gath(2, 2)
        proc(0, 0, 3, first=True)

        def quad(g, _):
            k = 4 * g + 1
            proc(k, 1, 0)
            proc(k + 1, 2, 1)
            proc(k + 2, 3, 2)
            proc(k + 3, 0, 3)
            return 0
        lax.fori_loop(0, (_SGC - 4) // 4, quad, 0)
        proc(_SGC - 3, 1, 0)
        proc(_SGC - 2, 2, 1)
        proc(_SGC - 1, 3, 2)
        wait_s(3)
        wait_g(0)
        wait_g(1)
        wait_g(2)
        return 0
    lax.fori_loop(0, _NST, stage, 0)

    plsc.subcore_barrier()
    for t in range(_RT // 128):
        pltpu.sync_copy(accS.at[pl.ds(r0 + t * 128, 128)],
                        acc_out.at[c, pl.ds(r0 + t * 128, 128)])


@functools.cache
def _sc_pass():
    return functools.partial(
        pl.kernel,
        out_type=jax.ShapeDtypeStruct((2, _NP, 128), jnp.float32),
        mesh=_sc_mesh(),
        compiler_params=pltpu.CompilerParams(needs_layout_passes=False),
        scratch_types=[
            pltpu.VMEM((_SGC, _CW), jnp.int32),
            pltpu.VMEM((_SGC, _CW), jnp.int32),
            pltpu.VMEM((_SGC, _CW), jnp.float32),
            pltpu.VMEM((_CW, 128), jnp.float32),
            pltpu.VMEM((_CW, 128), jnp.float32),
            pltpu.VMEM((_CW, 128), jnp.float32),
            pltpu.VMEM((_CW, 128), jnp.float32),
            pltpu.VMEM_SHARED((_NP, 128), jnp.float32),
            pltpu.SemaphoreType.DMA, pltpu.SemaphoreType.DMA,
            pltpu.SemaphoreType.DMA, pltpu.SemaphoreType.DMA,
            pltpu.SemaphoreType.DMA, pltpu.SemaphoreType.DMA,
            pltpu.SemaphoreType.DMA, pltpu.SemaphoreType.DMA,
        ],
    )(_sc_pass_body)


def _mm_body(x_ref, w_ref, o_ref):
    r = jnp.dot(x_ref[...], w_ref[...], preferred_element_type=jnp.float32)
    o_ref[0] = r[:, :128]
    o_ref[1] = r[:, 128:]


def _mid_body(a_ref, xl_ref, dv_ref, b_ref, w_ref, o_ref):
    accv = jnp.concatenate([a_ref[0], a_ref[1]], axis=1)
    xlv = jnp.concatenate([xl_ref[0], xl_ref[1]], axis=1)
    dv = dv_ref[...]
    h = jax.nn.relu(accv + (dv * dv) * xlv + b_ref[...])
    r = jnp.dot(h, w_ref[...], preferred_element_type=jnp.float32)
    o_ref[0] = r[:, :128]
    o_ref[1] = r[:, 128:]


def _head_body(a2, xl2, dv, c2b, bng, bnb, edg, f0w, beg, beb, dgp,
               embp, bdg, bdb, f2w, f2b,
               o_emb,
               ssum, ssq, cnt, esum, esq, sc1, sh1, eA, eB, T3):
    p = pl.program_id(0)
    i = pl.program_id(1)
    dvv = dv[...]
    out2 = (jnp.concatenate([a2[0], a2[1]], axis=1)
            + (dvv * dvv) * jnp.concatenate([xl2[0], xl2[1]], axis=1)
            + c2b[...])
    oh = (dgp[...] == lax.broadcasted_iota(jnp.int32, (_BLK, 512), 1))
    oh = oh.astype(jnp.float32)

    @pl.when(p == 0)
    def _():
        bsum = jnp.sum(out2, axis=0, keepdims=True)
        bsq = jnp.sum(out2 * out2, axis=0, keepdims=True)
        bcnt = jnp.sum(oh, axis=0, keepdims=True)
        eb = edg[...]
        bes = jnp.sum(eb)
        bes2 = jnp.sum(eb * eb)

        @pl.when(i == 0)
        def _():
            ssum[...] = bsum
            ssq[...] = bsq
            cnt[...] = bcnt
            esum[...] = jnp.full((1, 1), bes, jnp.float32)
            esq[...] = jnp.full((1, 1), bes2, jnp.float32)

        @pl.when(i > 0)
        def _():
            ssum[...] += bsum
            ssq[...] += bsq
            cnt[...] += bcnt
            esum[...] += jnp.full((1, 1), bes, jnp.float32)
            esq[...] += jnp.full((1, 1), bes2, jnp.float32)

        o_emb[...] = out2

    @pl.when(p == 1)
    def _():
        @pl.when(i == 0)
        def _():
            m = ssum[...] / float(_N)
            v = ssq[...] / float(_N) - m * m
            s1 = bng[...] * lax.rsqrt(v + 1e-5)
            sc1[...] = s1
            sh1[...] = bnb[...] - m * s1
            me = esum[...] / float(_N)
            ve = esq[...] / float(_N) - me * me
            se = lax.rsqrt(ve * (f0w[...] * f0w[...]) + 1e-5)
            A = f0w[...] * beg[...] * se
            eA[...] = A
            eB[...] = beb[...] - me * A
            cn = cnt[...] / float(_N)
            emv = embp[...]
            md = jnp.dot(cn, emv, preferred_element_type=jnp.float32)
            vd = jnp.dot(cn, emv * emv,
                         preferred_element_type=jnp.float32) - md * md
            sd = bdg[...] * lax.rsqrt(vd + 1e-5)
            T = jax.nn.relu(emv * sd + (bdb[...] - md * sd))
            T3[...] = jnp.dot(T, f2w[512:768, :],
                              preferred_element_type=jnp.float32)

        h1 = jax.nn.relu(out2 * sc1[...] + sh1[...])
        e1 = jax.nn.relu(edg[...] * eA[...] + eB[...])
        cat = (jnp.dot(h1, f2w[0:256, :], preferred_element_type=jnp.float32)
               + jnp.dot(e1, f2w[256:512, :],
                         preferred_element_type=jnp.float32)
               + jnp.dot(oh, T3[...], preferred_element_type=jnp.float32)
               + f2b[...])
        nr = jnp.sqrt(jnp.sum(cat * cat, axis=1, keepdims=True))
        o_emb[...] = cat / jnp.maximum(nr, 1e-12)


def _max_body(d_ref, o_ref):
    o_ref[...] = jnp.full((1, 1), jnp.max(d_ref[...]), jnp.float32)


def _log_body(q_ref, qu_ref, qw_ref, pi_ref, mx_ref, lp_ref, ln_ref):
    q = q_ref[...]
    oh = (pi_ref[...] == lax.broadcasted_iota(jnp.int32, (_BATCH, 512), 1))
    w = jnp.dot(oh.astype(jnp.float32), qw_ref[...],
                preferred_element_type=jnp.float32)
    w = jax.nn.sigmoid(w / mx_ref[0, 0] * 16.0 - 6.0)
    ln_ref[...] = jnp.dot(q, qu_ref[...],
                          preferred_element_type=jnp.float32) * w * 100.0
    lp_ref[...] = jnp.sum(q * q, axis=1, keepdims=True) * 100.0


def kernel(idx, x, edge_index, edge_weight, edge, degree, batch, dist, perm,
           c1W, c1b, c2W, c2b, bn_g, bn_b, be_g, be_b, bd_g, bd_b,
           f0W, f0b, emb, f2W, f2b, queue, queue_w):
    f32 = jnp.float32
    src = edge_index[0]
    dst = edge_index[1]
    pe = _EP - _E
    src_p = jnp.concatenate([src, jnp.zeros((pe,), jnp.int32)]).reshape(_NCH, _CW)
    dst_p = jnp.concatenate([dst, jnp.zeros((pe,), jnp.int32)]).reshape(_NCH, _CW)
    ew_p = jnp.concatenate([edge_weight, jnp.zeros((pe,), f32)]).reshape(_NCH, _CW)

    dinv, nrm = _sc_prep()(src_p, dst_p, ew_p)
    xl1 = pl.pallas_call(
        _mm_body,
        grid=(_NB,),
        in_specs=[pl.BlockSpec((_BLK, _D), lambda i: (i, 0)),
                  pl.BlockSpec((_D, _D), lambda i: (0, 0))],
        out_specs=pl.BlockSpec((2, _BLK, 128), lambda i: (0, i, 0)),
        out_shape=jax.ShapeDtypeStruct((2, _NP, 128), f32),
    )(x, c1W)

    acc1 = _sc_pass()(src_p, dst_p, nrm, xl1)

    dinv2d = dinv.reshape(_NP, 1)
    xl2 = pl.pallas_call(
        _mid_body,
        grid=(_NB,),
        in_specs=[pl.BlockSpec((2, _BLK, 128), lambda i: (0, i, 0)),
                  pl.BlockSpec((2, _BLK, 128), lambda i: (0, i, 0)),
                  pl.BlockSpec((_BLK, 1), lambda i: (i, 0)),
                  pl.BlockSpec((1, _D), lambda i: (0, 0)),
                  pl.BlockSpec((_D, _D), lambda i: (0, 0))],
        out_specs=pl.BlockSpec((2, _BLK, 128), lambda i: (0, i, 0)),
        out_shape=jax.ShapeDtypeStruct((2, _NP, 128), f32),
    )(acc1, xl1, dinv2d, c1b.reshape(1, _D), c2W)

    acc2 = _sc_pass()(src_p, dst_p, nrm, xl2)

    emb_p = jnp.pad(emb, ((0, 512 - _NN), (0, 0)))

    maxd = pl.pallas_call(
        _max_body,
        out_shape=jax.ShapeDtypeStruct((1, 1), f32),
    )(dist)

    full1 = pl.BlockSpec((1, _D), lambda p, i: (0, 0))
    embs = pl.pallas_call(
        _head_body,
        grid=(2, _NB),
        in_specs=[pl.BlockSpec((2, _BLK, 128), lambda p, i: (0, i, 0)),
                  pl.BlockSpec((2, _BLK, 128), lambda p, i: (0, i, 0)),
                  pl.BlockSpec((_BLK, 1), lambda p, i: (i, 0)),
                  full1, full1, full1,
                  pl.BlockSpec((_BLK, 1), lambda p, i: (i, 0)),
                  full1, full1, full1,
                  pl.BlockSpec((_BLK, 1), lambda p, i: (i, 0)),
                  pl.BlockSpec((512, _D), lambda p, i: (0, 0)),
                  full1, full1,
                  pl.BlockSpec((768, _D), lambda p, i: (0, 0)),
                  full1],
        out_specs=pl.BlockSpec((_BLK, _D), lambda p, i: (i, 0)),
        out_shape=jax.ShapeDtypeStruct((_N, _D), f32),
        scratch_shapes=[pltpu.VMEM((1, _D), f32), pltpu.VMEM((1, _D), f32),
                        pltpu.VMEM((1, 512), f32),
                        pltpu.VMEM((1, 1), f32), pltpu.VMEM((1, 1), f32),
                        pltpu.VMEM((1, _D), f32), pltpu.VMEM((1, _D), f32),
                        pltpu.VMEM((1, _D), f32), pltpu.VMEM((1, _D), f32),
                        pltpu.VMEM((512, _D), f32)],
    )(acc2, xl2, dinv2d, c2b.reshape(1, _D), bn_g.reshape(1, _D),
      bn_b.reshape(1, _D), edge.reshape(_N, 1), f0W.reshape(1, _D),
      be_g.reshape(1, _D), be_b.reshape(1, _D), degree.reshape(_N, 1),
      emb_p, bd_g.reshape(1, _D), bd_b.reshape(1, _D), f2W,
      f2b.reshape(1, _D))

    start = idx * batch
    qm = lax.dynamic_slice_in_dim(embs, start, _BATCH, 0)
    pidx = lax.dynamic_slice_in_dim(perm, start, _BATCH, 0)
    qw_p = jnp.pad(queue_w, ((0, 512 - _NN), (0, 0)))

    lp, ln = pl.pallas_call(
        _log_body,
        out_shape=[jax.ShapeDtypeStruct((_BATCH, 1), f32),
                   jax.ShapeDtypeStruct((_BATCH, _K), f32)],
    )(qm, queue, qw_p, pidx.reshape(_BATCH, 1), maxd)

    logits = jnp.concatenate([lp, ln], axis=1)
    labels = jnp.zeros((_BATCH,), dtype=jnp.int32)
    return (embs, logits, labels)


# speedup vs baseline: 1.1387x; 1.1387x over previous
import functools

import jax
import jax.numpy as jnp
from jax import lax
from jax.experimental import pallas as pl
from jax.experimental.pallas import tpu as pltpu
from jax.experimental.pallas import tpu_sc as plsc

_N = 10000
_E = 160000
_D = 256
_NN = 500
_K = 256
_BATCH = 256

_NC, _NS, _L = 2, 16, 16
_NP = 10240
_ECH = 1280
_EP = _ECH * 128
_PTC = _ECH // _NS
_RT = _NP // _NS
_BLK = 2000
_NB = _N // _BLK
_RTO = _N // _NS

@functools.cache
def _sc_mesh():
    return plsc.VectorSubcoreMesh(core_axis_name="c", subcore_axis_name="s",
                                  num_cores=_NC, num_subcores=_NS)


def _nr_rsqrt(x):
    i = plsc.bitcast(x, jnp.int32)
    i = jnp.int32(0x5F3759DF) - (i >> 1)
    y = plsc.bitcast(i, jnp.float32)
    for _ in range(3):
        y = y * (1.5 - 0.5 * x * y * y)
    return y


def _sc_prep_body(src2d, dst2d, ew2d, dinv_out, norm_out,
                  degt, srct, dstt, ewt, rbuf, dbuf, dinvt, nrmt,
                  slab, sdinv):
    c = lax.axis_index("c")
    s = lax.axis_index("s")
    pltpu.sync_copy(dst2d.at[pl.ds(s * _PTCH, _PTCH)], dstt)
    pltpu.sync_copy(ew2d.at[pl.ds(s * _PTCH, _PTCH)], ewt)
    pltpu.sync_copy(src2d.at[pl.ds(s * _PTCH, _PTCH)], srct)

    def zero(i, _):
        degt[pl.ds(i * 16, 16)] = jnp.zeros((16,), jnp.float32)
        return 0
    lax.fori_loop(0, _NP // 16, zero, 0)

    def scat(k, _):
        for g in range(_CW // 16):
            d16 = dstt[k, pl.ds(g * 16, 16)]
            w16 = ewt[k, pl.ds(g * 16, 16)]
            plsc.addupdate_scatter(degt, [d16], w16)
        return 0
    lax.fori_loop(0, _PTCH, scat, 0)

    pltpu.sync_copy(degt, slab.at[s])
    plsc.subcore_barrier()

    r0 = s * _RT
    for p in range(_NS):
        pltpu.sync_copy(slab.at[p, pl.ds(r0, _RT)], rbuf.at[p])

    def red(v, _):
        acc = rbuf[0, pl.ds(v * 16, 16)]
        for p in range(1, _NS):
            acc = acc + rbuf[p, pl.ds(v * 16, 16)]
        acc = acc + 1.0
        dbuf[pl.ds(v * 16, 16)] = _nr_rsqrt(acc)
        return 0
    lax.fori_loop(0, _RT // 16, red, 0)

    pltpu.sync_copy(dbuf, sdinv.at[pl.ds(r0, _RT)])

    @pl.when(c == 0)
    def _():
        pltpu.sync_copy(dbuf, dinv_out.at[pl.ds(r0, _RT)])

    plsc.subcore_barrier()
    pltpu.sync_copy(sdinv, dinvt)

    def nrm(k, _):
        for g in range(_CW // 16):
            s16 = srct[k, pl.ds(g * 16, 16)]
            d16 = dstt[k, pl.ds(g * 16, 16)]
            w16 = ewt[k, pl.ds(g * 16, 16)]
            a = plsc.load_gather(dinvt, [s16])
            b = plsc.load_gather(dinvt, [d16])
            nrmt[k, pl.ds(g * 16, 16)] = a * w16 * b
        return 0
    lax.fori_loop(0, _PTCH, nrm, 0)

    @pl.when(c == 0)
    def _():
        pltpu.sync_copy(nrmt, norm_out.at[pl.ds(s * _PTCH, _PTCH)])


@functools.cache
def _sc_prep():
    return functools.partial(
        pl.kernel,
        out_type=(jax.ShapeDtypeStruct((_NP,), jnp.float32),
                  jax.ShapeDtypeStruct((_NCH, _CW), jnp.float32)),
        mesh=_sc_mesh(),
        compiler_params=pltpu.CompilerParams(needs_layout_passes=False),
        scratch_types=[
            pltpu.VMEM((_NP,), jnp.float32),
            pltpu.VMEM((_PTCH, _CW), jnp.int32),
            pltpu.VMEM((_PTCH, _CW), jnp.int32),
            pltpu.VMEM((_PTCH, _CW), jnp.float32),
            pltpu.VMEM((_NS, _RT), jnp.float32),
            pltpu.VMEM((_RT,), jnp.float32),
            pltpu.VMEM((_NP,), jnp.float32),
            pltpu.VMEM((_PTCH, _CW), jnp.float32),
            pltpu.VMEM_SHARED((_NS, _NP), jnp.float32),
            pltpu.VMEM_SHARED((_NP,), jnp.float32),
        ],
    )(_sc_prep_body)


_CW = 128
_NCH = _EP // _CW
_PTCH = _NCH // _NS
_SGC = 40
_NST = _PTCH // _SGC


def _sc_pass_body(src2d, dst2d, nrm2d, xlh, acc_out,
                  srct, dstt, nrmt, gb0, gb1, accS,
                  sem0, sem1, ssc0, ssc1):
    c = lax.axis_index("c")
    s = lax.axis_index("s")

    def zero(i, _):
        for j in range(8):
            gb0[i, pl.ds(j * 16, 16)] = jnp.zeros((16,), jnp.float32)
        return 0
    lax.fori_loop(0, 128, zero, 0)
    r0 = s * _RT
    for t in range(_RT // 128):
        pltpu.sync_copy(gb0, accS.at[pl.ds(r0 + t * 128, 128)])
    plsc.subcore_barrier()

    xl = xlh.at[c]

    def scale(buf, k):
        def grp(g, _):
            n16 = nrmt[k, pl.ds(g * 16, 16)]
            for e in range(16):
                nb = n16.at[jnp.full((16,), e, jnp.int32)].get(
                    mode="promise_in_bounds")
                row = g * 16 + e
                for j in range(8):
                    buf[row, pl.ds(j * 16, 16)] = (
                        buf[row, pl.ds(j * 16, 16)] * nb)
            return 0
        lax.fori_loop(0, _CW // 16, grp, 0)

    def gath(k, buf, sem):
        pltpu.async_copy(xl.at[srct.at[k]], buf, sem)

    def wait_g(buf, sem):
        pltpu.make_async_copy(xl.at[srct.at[0]], buf, sem).wait()

    def scat(k, buf, sem):
        pltpu.async_copy(buf, accS.at[dstt.at[k]], sem, add=True)

    def wait_s(buf, sem):
        pltpu.make_async_copy(buf, accS.at[dstt.at[0]], sem).wait()

    def stage(st, _):
        g0 = s * _PTCH + st * _SGC
        pltpu.sync_copy(src2d.at[pl.ds(g0, _SGC)], srct)
        pltpu.sync_copy(dst2d.at[pl.ds(g0, _SGC)], dstt)
        pltpu.sync_copy(nrm2d.at[pl.ds(g0, _SGC)], nrmt)
        gath(0, gb0, sem0)
        gath(1, gb1, sem1)
        wait_g(gb0, sem0)
        scale(gb0, 0)
        scat(0, gb0, ssc0)

        def pair(g, _):
            ko = 2 * g + 1
            wait_s(gb0, ssc0)
            gath(ko + 1, gb0, sem0)
            wait_g(gb1, sem1)
            scale(gb1, ko)
            scat(ko, gb1, ssc1)
            wait_s(gb1, ssc1)
            gath(ko + 2, gb1, sem1)
            wait_g(gb0, sem0)
            scale(gb0, ko + 1)
            scat(ko + 1, gb0, ssc0)
            return 0
        lax.fori_loop(0, (_SGC - 2) // 2, pair, 0)
        wait_s(gb0, ssc0)
        wait_g(gb1, sem1)
        scale(gb1, _SGC - 1)
        scat(_SGC - 1, gb1, ssc1)
        wait_s(gb1, ssc1)
        return 0
    lax.fori_loop(0, _NST, stage, 0)

    plsc.subcore_barrier()
    for t in range(_RT // 128):
        pltpu.sync_copy(accS.at[pl.ds(r0 + t * 128, 128)],
                        acc_out.at[c, pl.ds(r0 + t * 128, 128)])


@functools.cache
def _sc_pass():
    return functools.partial(
        pl.kernel,
        out_type=jax.ShapeDtypeStruct((2, _NP, 128), jnp.float32),
        mesh=_sc_mesh(),
        compiler_params=pltpu.CompilerParams(needs_layout_passes=False),
        scratch_types=[
            pltpu.VMEM((_SGC, _CW), jnp.int32),
            pltpu.VMEM((_SGC, _CW), jnp.int32),
            pltpu.VMEM((_SGC, _CW), jnp.float32),
            pltpu.VMEM((_CW, 128), jnp.float32),
            pltpu.VMEM((_CW, 128), jnp.float32),
            pltpu.VMEM_SHARED((_NP, 128), jnp.float32),
            pltpu.SemaphoreType.DMA, pltpu.SemaphoreType.DMA,
            pltpu.SemaphoreType.DMA, pltpu.SemaphoreType.DMA,
        ],
    )(_sc_pass_body)


def _mm_body(x_ref, w_ref, o_ref):
    r = jnp.dot(x_ref[...], w_ref[...], preferred_element_type=jnp.float32)
    o_ref[0] = r[:, :128]
    o_ref[1] = r[:, 128:]


def _mid_body(a_ref, xl_ref, dv_ref, b_ref, w_ref, o_ref):
    accv = jnp.concatenate([a_ref[0], a_ref[1]], axis=1)
    xlv = jnp.concatenate([xl_ref[0], xl_ref[1]], axis=1)
    dv = dv_ref[...]
    h = jax.nn.relu(accv + (dv * dv) * xlv + b_ref[...])
    r = jnp.dot(h, w_ref[...], preferred_element_type=jnp.float32)
    o_ref[0] = r[:, :128]
    o_ref[1] = r[:, 128:]


def _head_body(a2, xl2, dv, c2b, bng, bnb, edg, f0w, beg, beb, dgp,
               embp, bdg, bdb, f2w, f2b,
               o_emb,
               ssum, ssq, cnt, esum, esq, sc1, sh1, eA, eB, T3):
    p = pl.program_id(0)
    i = pl.program_id(1)
    dvv = dv[...]
    out2 = (jnp.concatenate([a2[0], a2[1]], axis=1)
            + (dvv * dvv) * jnp.concatenate([xl2[0], xl2[1]], axis=1)
            + c2b[...])
    oh = (dgp[...] == lax.broadcasted_iota(jnp.int32, (_BLK, 512), 1))
    oh = oh.astype(jnp.float32)

    @pl.when(p == 0)
    def _():
        bsum = jnp.sum(out2, axis=0, keepdims=True)
        bsq = jnp.sum(out2 * out2, axis=0, keepdims=True)
        bcnt = jnp.sum(oh, axis=0, keepdims=True)
        eb = edg[...]
        bes = jnp.sum(eb)
        bes2 = jnp.sum(eb * eb)

        @pl.when(i == 0)
        def _():
            ssum[...] = bsum
            ssq[...] = bsq
            cnt[...] = bcnt
            esum[...] = jnp.full((1, 1), bes, jnp.float32)
            esq[...] = jnp.full((1, 1), bes2, jnp.float32)

        @pl.when(i > 0)
        def _():
            ssum[...] += bsum
            ssq[...] += bsq
            cnt[...] += bcnt
            esum[...] += jnp.full((1, 1), bes, jnp.float32)
            esq[...] += jnp.full((1, 1), bes2, jnp.float32)

        o_emb[...] = out2

    @pl.when(p == 1)
    def _():
        @pl.when(i == 0)
        def _():
            m = ssum[...] / float(_N)
            v = ssq[...] / float(_N) - m * m
            s1 = bng[...] * lax.rsqrt(v + 1e-5)
            sc1[...] = s1
            sh1[...] = bnb[...] - m * s1
            me = esum[...] / float(_N)
            ve = esq[...] / float(_N) - me * me
            se = lax.rsqrt(ve * (f0w[...] * f0w[...]) + 1e-5)
            A = f0w[...] * beg[...] * se
            eA[...] = A
            eB[...] = beb[...] - me * A
            cn = cnt[...] / float(_N)
            emv = embp[...]
            md = jnp.dot(cn, emv, preferred_element_type=jnp.float32)
            vd = jnp.dot(cn, emv * emv,
                         preferred_element_type=jnp.float32) - md * md
            sd = bdg[...] * lax.rsqrt(vd + 1e-5)
            T = jax.nn.relu(emv * sd + (bdb[...] - md * sd))
            T3[...] = jnp.dot(T, f2w[512:768, :],
                              preferred_element_type=jnp.float32)

        h1 = jax.nn.relu(out2 * sc1[...] + sh1[...])
        e1 = jax.nn.relu(edg[...] * eA[...] + eB[...])
        cat = (jnp.dot(h1, f2w[0:256, :], preferred_element_type=jnp.float32)
               + jnp.dot(e1, f2w[256:512, :],
                         preferred_element_type=jnp.float32)
               + jnp.dot(oh, T3[...], preferred_element_type=jnp.float32)
               + f2b[...])
        nr = jnp.sqrt(jnp.sum(cat * cat, axis=1, keepdims=True))
        o_emb[...] = cat / jnp.maximum(nr, 1e-12)


def _max_body(d_ref, o_ref):
    o_ref[...] = jnp.full((1, 1), jnp.max(d_ref[...]), jnp.float32)


def _log_body(q_ref, qu_ref, qw_ref, pi_ref, mx_ref, lp_ref, ln_ref):
    q = q_ref[...]
    oh = (pi_ref[...] == lax.broadcasted_iota(jnp.int32, (_BATCH, 512), 1))
    w = jnp.dot(oh.astype(jnp.float32), qw_ref[...],
                preferred_element_type=jnp.float32)
    w = jax.nn.sigmoid(w / mx_ref[0, 0] * 16.0 - 6.0)
    ln_ref[...] = jnp.dot(q, qu_ref[...],
                          preferred_element_type=jnp.float32) * w * 100.0
    lp_ref[...] = jnp.sum(q * q, axis=1, keepdims=True) * 100.0


def kernel(idx, x, edge_index, edge_weight, edge, degree, batch, dist, perm,
           c1W, c1b, c2W, c2b, bn_g, bn_b, be_g, be_b, bd_g, bd_b,
           f0W, f0b, emb, f2W, f2b, queue, queue_w):
    f32 = jnp.float32
    src = edge_index[0]
    dst = edge_index[1]
    pe = _EP - _E
    src_p = jnp.concatenate([src, jnp.zeros((pe,), jnp.int32)]).reshape(_NCH, _CW)
    dst_p = jnp.concatenate([dst, jnp.zeros((pe,), jnp.int32)]).reshape(_NCH, _CW)
    ew_p = jnp.concatenate([edge_weight, jnp.zeros((pe,), f32)]).reshape(_NCH, _CW)

    dinv, nrm = _sc_prep()(src_p, dst_p, ew_p)
    xl1 = pl.pallas_call(
        _mm_body,
        grid=(_NB,),
        in_specs=[pl.BlockSpec((_BLK, _D), lambda i: (i, 0)),
                  pl.BlockSpec((_D, _D), lambda i: (0, 0))],
        out_specs=pl.BlockSpec((2, _BLK, 128), lambda i: (0, i, 0)),
        out_shape=jax.ShapeDtypeStruct((2, _NP, 128), f32),
    )(x, c1W)

    acc1 = _sc_pass()(src_p, dst_p, nrm, xl1)

    dinv2d = dinv.reshape(_NP, 1)
    xl2 = pl.pallas_call(
        _mid_body,
        grid=(_NB,),
        in_specs=[pl.BlockSpec((2, _BLK, 128), lambda i: (0, i, 0)),
                  pl.BlockSpec((2, _BLK, 128), lambda i: (0, i, 0)),
                  pl.BlockSpec((_BLK, 1), lambda i: (i, 0)),
                  pl.BlockSpec((1, _D), lambda i: (0, 0)),
                  pl.BlockSpec((_D, _D), lambda i: (0, 0))],
        out_specs=pl.BlockSpec((2, _BLK, 128), lambda i: (0, i, 0)),
        out_shape=jax.ShapeDtypeStruct((2, _NP, 128), f32),
    )(acc1, xl1, dinv2d, c1b.reshape(1, _D), c2W)

    acc2 = _sc_pass()(src_p, dst_p, nrm, xl2)

    emb_p = jnp.pad(emb, ((0, 512 - _NN), (0, 0)))

    maxd = pl.pallas_call(
        _max_body,
        out_shape=jax.ShapeDtypeStruct((1, 1), f32),
    )(dist)

    full1 = pl.BlockSpec((1, _D), lambda p, i: (0, 0))
    embs = pl.pallas_call(
        _head_body,
        grid=(2, _NB),
        in_specs=[pl.BlockSpec((2, _BLK, 128), lambda p, i: (0, i, 0)),
                  pl.BlockSpec((2, _BLK, 128), lambda p, i: (0, i, 0)),
                  pl.BlockSpec((_BLK, 1), lambda p, i: (i, 0)),
                  full1, full1, full1,
                  pl.BlockSpec((_BLK, 1), lambda p, i: (i, 0)),
                  full1, full1, full1,
                  pl.BlockSpec((_BLK, 1), lambda p, i: (i, 0)),
                  pl.BlockSpec((512, _D), lambda p, i: (0, 0)),
                  full1, full1,
                  pl.BlockSpec((768, _D), lambda p, i: (0, 0)),
                  full1],
        out_specs=pl.BlockSpec((_BLK, _D), lambda p, i: (i, 0)),
        out_shape=jax.ShapeDtypeStruct((_N, _D), f32),
        scratch_shapes=[pltpu.VMEM((1, _D), f32), pltpu.VMEM((1, _D), f32),
                        pltpu.VMEM((1, 512), f32),
                        pltpu.VMEM((1, 1), f32), pltpu.VMEM((1, 1), f32),
                        pltpu.VMEM((1, _D), f32), pltpu.VMEM((1, _D), f32),
                        pltpu.VMEM((1, _D), f32), pltpu.VMEM((1, _D), f32),
                        pltpu.VMEM((512, _D), f32)],
    )(acc2, xl2, dinv2d, c2b.reshape(1, _D), bn_g.reshape(1, _D),
      bn_b.reshape(1, _D), edge.reshape(_N, 1), f0W.reshape(1, _D),
      be_g.reshape(1, _D), be_b.reshape(1, _D), degree.reshape(_N, 1),
      emb_p, bd_g.reshape(1, _D), bd_b.reshape(1, _D), f2W,
      f2b.reshape(1, _D))

    start = idx * batch
    qm = lax.dynamic_slice_in_dim(embs, start, _BATCH, 0)
    pidx = lax.dynamic_slice_in_dim(perm, start, _BATCH, 0)
    qw_p = jnp.pad(queue_w, ((0, 512 - _NN), (0, 0)))

    lp, ln = pl.pallas_call(
        _log_body,
        out_shape=[jax.ShapeDtypeStruct((_BATCH, 1), f32),
                   jax.ShapeDtypeStruct((_BATCH, _K), f32)],
    )(qm, queue, qw_p, pidx.reshape(_BATCH, 1), maxd)

    logits = jnp.concatenate([lp, ln], axis=1)
    labels = jnp.zeros((_BATCH,), dtype=jnp.int32)
    return (embs, logits, labels)
